# baseline (speedup 1.0000x reference)
import functools

import jax
import jax.numpy as jnp
from jax import lax
from jax.experimental import pallas as pl
from jax.experimental.pallas import tpu as pltpu
from jax.experimental.pallas import tpu_sc as plsc

NT = 4
NBIN = 400
HALO = 16


@functools.lru_cache(maxsize=None)
def _build_sc_hist(L: int, D: int):
    assert D == 16, "kernel assumes D == SC lane width (16)"
    NW = 32
    C = L // NW
    B = 2048
    NBLK = C // B
    NTL = B // 128 + 1
    SQL = B + 128
    TW = NTL * 1024
    assert C % B == 0 and L % NW == 0 and B % 128 == 0 and L % 128 == 0
    ACC = NT * NBIN * D

    mesh = plsc.VectorSubcoreMesh(core_axis_name="c", subcore_axis_name="s")

    @functools.partial(
        pl.kernel,
        mesh=mesh,
        compiler_params=pltpu.CompilerParams(use_tc_tiling_on_sc=False,
                                             needs_layout_passes=False),
        out_type=jax.ShapeDtypeStruct((NW, ACC), jnp.float32),
        scratch_types=[
            pltpu.VMEM((ACC,), jnp.float32),
            pltpu.VMEM((D // 8, TW), jnp.float32),
            pltpu.VMEM((SQL,), jnp.int32),
            pltpu.VMEM((D // 8, TW), jnp.float32),
            pltpu.VMEM((SQL,), jnp.int32),
            pltpu.VMEM((D // 8, 1024), jnp.float32),
            pltpu.VMEM((2 * HALO,), jnp.int32),
            pltpu.SemaphoreType.DMA,
            pltpu.SemaphoreType.DMA,
        ],
    )
    def sc_hist(seq_hbm, embt_hbm, out_hbm, acc,
                embv0, seqv0, embv1, seqv1, temb, tseq, sem0, sem1):
        wid = lax.axis_index("s") * 2 + lax.axis_index("c")
        lane = lax.iota(jnp.int32, 16)
        dt_vec = lane >> 3
        rof_vec = (lane & 7) * 128

        zero = jnp.zeros((D,), jnp.float32)

        def zero_body(j, carry):
            acc[pl.ds(pl.multiple_of(j * D, D), D)] = zero
            return carry

        lax.fori_loop(0, ACC // D, zero_body, None)

        wbase = wid * C

        def dma_base(b):
            base = wbase + b * B
            edge = base >= L - B
            ofs = jnp.where(edge, 128, 0)
            dbase = pl.multiple_of(base - ofs, 128)
            ng = jnp.where(edge, B // 16 - 1, B // 16)
            return dbase, ofs, ng

        def issue(b, embv, seqv, sem):
            dbase, _, _ = dma_base(b)
            pltpu.async_copy(seq_hbm.at[pl.ds(dbase, SQL)], seqv, sem)
            pltpu.async_copy(
                embt_hbm.at[:, pl.ds(pl.multiple_of(dbase * 8, 1024), TW)],
                embv, sem)

        def drain(embv, seqv, sem):
            pltpu.make_async_copy(seq_hbm.at[pl.ds(0, SQL)], seqv, sem).wait()
            pltpu.make_async_copy(embt_hbm.at[:, pl.ds(0, TW)], embv,
                                  sem).wait()

        def compute(b, embv, seqv):
            _, ofs, ng = dma_base(b)

            @plsc.parallel_loop(0, ng, unroll=2)
            def grp_body(g):
                i0 = pl.multiple_of(g * 16 + ofs, 16)
                sA = seqv[pl.ds(i0, 16)]
                sA320 = sA * (20 * D)
                ib0 = rof_vec + jnp.broadcast_to(
                    ((i0 >> 7) << 10) + (i0 & 127), (16,))
                i1 = i0 + 16
                ib1 = rof_vec + jnp.broadcast_to(
                    ((i1 >> 7) << 10) + (i1 & 127), (16,))
                rows = [plsc.load_gather(embv, [dt_vec, ib0 + j])
                        for j in range(16)]
                rows += [plsc.load_gather(embv, [dt_vec, ib1 + j])
                         for j in range(NT)]
                for t in range(NT):
                    sB = seqv[pl.ds(i0 + t + 1, 16)]
                    offv = sA320 + sB * D + (t * NBIN * D)
                    for j in range(16):
                        off = pl.multiple_of(offv[j], D)
                        plsc.addupdate(acc.at[pl.ds(off, D)],
                                       rows[j] + rows[j + t + 1])

        issue(0, embv0, seqv0, sem0)

        def pair_body(h, carry):
            b0 = h * 2
            drain(embv0, seqv0, sem0)
            issue(b0 + 1, embv1, seqv1, sem1)
            compute(b0, embv0, seqv0)
            drain(embv1, seqv1, sem1)

            @pl.when(h < NBLK // 2 - 1)
            def _prefetch_next():
                issue(b0 + 2, embv0, seqv0, sem0)

            compute(b0 + 1, embv1, seqv1)
            return carry

        lax.fori_loop(0, NBLK // 2, pair_body, None)

        @pl.when(wid == NW - 1)
        def _edge_tail():
            tbase = L - 2 * HALO
            pltpu.sync_copy(seq_hbm.at[pl.ds(tbase, 2 * HALO)], tseq)
            pltpu.sync_copy(embt_hbm.at[:, pl.ds(L * 8 - 1024, 1024)], temb)
            sT = tseq[pl.ds(HALO, 16)]

            def trow(li):
                idx = rof_vec + (96 + li)
                return plsc.load_gather(temb, [dt_vec, idx])

            for t in range(NT):
                for li in range(HALO, 2 * HALO - t - 1):
                    sa = sT[li - HALO]
                    sb = sT[li + t + 1 - HALO]
                    off = (t * NBIN + sa * 20 + sb) * D
                    v = trow(li) + trow(li + t + 1)
                    plsc.addupdate(acc.at[pl.ds(pl.multiple_of(off, D), D)],
                                   v)

        pltpu.sync_copy(acc, out_hbm.at[wid])

    return sc_hist


def kernel(query_seq, emb, k):
    L = query_seq.shape[0]
    D = emb.shape[-1]
    seq32 = query_seq.astype(jnp.int32)
    embt = (emb.T.reshape(D // 8, 8, L // 128, 128)
            .transpose(0, 2, 1, 3).reshape(D // 8, L * 8))
    partials = _build_sc_hist(L, D)(seq32, embt)
    hist = partials.sum(axis=0).reshape(NT, NBIN, D)
    t = jnp.arange(NT)
    n = (L - t - 1).astype(jnp.float32)
    gate = (t <= k).astype(jnp.float32)
    out = hist * (0.5 * gate / n)[:, None, None]
    return out.reshape(NT, 20, 20, D)


# speedup vs baseline: 1.4572x; 1.4572x over previous
import functools

import jax
import jax.numpy as jnp
from jax import lax
from jax.experimental import pallas as pl
from jax.experimental.pallas import tpu as pltpu
from jax.experimental.pallas import tpu_sc as plsc

NT = 4
NBIN = 400
HALO = 16


@functools.lru_cache(maxsize=None)
def _build_sc_hist(L: int, D: int):
    assert D == 16, "kernel assumes D == SC lane width (16)"
    NW = 32
    C = L // NW
    B = 2048
    NBLK = C // B
    assert C % B == 0 and L % NW == 0 and B % 16 == 0
    ACC = NT * NBIN * D

    mesh = plsc.VectorSubcoreMesh(core_axis_name="c", subcore_axis_name="s")

    @functools.partial(
        pl.kernel,
        mesh=mesh,
        compiler_params=pltpu.CompilerParams(use_tc_tiling_on_sc=False,
                                             needs_layout_passes=False),
        out_type=jax.ShapeDtypeStruct((NW, ACC), jnp.float32),
        scratch_types=[
            pltpu.VMEM((ACC,), jnp.float32),
            pltpu.VMEM((D, B + HALO), jnp.float32),
            pltpu.VMEM((B + HALO,), jnp.int32),
            pltpu.VMEM((D, B + HALO), jnp.float32),
            pltpu.VMEM((B + HALO,), jnp.int32),
            pltpu.VMEM((D, 2 * HALO), jnp.float32),
            pltpu.VMEM((2 * HALO,), jnp.int32),
            pltpu.SemaphoreType.DMA,
            pltpu.SemaphoreType.DMA,
        ],
    )
    def sc_hist(seq_hbm, embt_hbm, out_hbm, acc,
                embv0, seqv0, embv1, seqv1, temb, tseq, sem0, sem1):
        wid = lax.axis_index("s") * 2 + lax.axis_index("c")
        lane = lax.iota(jnp.int32, 16)

        zero = jnp.zeros((D,), jnp.float32)

        def zero_body(j, carry):
            acc[pl.ds(pl.multiple_of(j * D, D), D)] = zero
            return carry

        lax.fori_loop(0, ACC // D, zero_body, None)

        wbase = wid * C

        def dma_base(b):
            base = wbase + b * B
            edge = base >= L - B
            ofs = jnp.where(edge, 16, 0)
            dbase = pl.multiple_of(base - ofs, 16)
            ng = jnp.where(edge, B // 16 - 1, B // 16)
            return dbase, ofs, ng

        def issue(b, embv, seqv, sem):
            dbase, _, _ = dma_base(b)
            pltpu.async_copy(seq_hbm.at[pl.ds(dbase, B + HALO)], seqv, sem)
            pltpu.async_copy(embt_hbm.at[:, pl.ds(dbase, B + HALO)], embv, sem)

        def drain(embv, seqv, sem):
            pltpu.make_async_copy(seq_hbm.at[pl.ds(0, B + HALO)], seqv,
                                  sem).wait()
            pltpu.make_async_copy(embt_hbm.at[:, pl.ds(0, B + HALO)], embv,
                                  sem).wait()

        def compute(b, embv, seqv):
            _, ofs, ng = dma_base(b)

            @plsc.parallel_loop(0, ng, unroll=2)
            def grp_body(g):
                i0 = pl.multiple_of(g * 16 + ofs, 16)
                sA = seqv[pl.ds(i0, 16)]
                sA320 = sA * (20 * D)
                rows = [plsc.load_gather(
                            embv, [lane, jnp.broadcast_to(i0 + j, (16,))])
                        for j in range(16 + NT)]
                for t in range(NT):
                    sB = seqv[pl.ds(i0 + t + 1, 16)]
                    offv = sA320 + sB * D + (t * NBIN * D)
                    for j in range(16):
                        off = pl.multiple_of(offv[j], D)
                        plsc.addupdate(acc.at[pl.ds(off, D)],
                                       rows[j] + rows[j + t + 1])

        issue(0, embv0, seqv0, sem0)

        def pair_body(h, carry):
            b0 = h * 2
            drain(embv0, seqv0, sem0)
            issue(b0 + 1, embv1, seqv1, sem1)
            compute(b0, embv0, seqv0)
            drain(embv1, seqv1, sem1)

            @pl.when(h < NBLK // 2 - 1)
            def _prefetch_next():
                issue(b0 + 2, embv0, seqv0, sem0)

            compute(b0 + 1, embv1, seqv1)
            return carry

        lax.fori_loop(0, NBLK // 2, pair_body, None)

        @pl.when(wid == NW - 1)
        def _edge_tail():
            tbase = L - 2 * HALO
            pltpu.sync_copy(seq_hbm.at[pl.ds(tbase, 2 * HALO)], tseq)
            pltpu.sync_copy(embt_hbm.at[:, pl.ds(tbase, 2 * HALO)], temb)
            sT = tseq[pl.ds(HALO, 16)]

            def trow(p):
                return plsc.load_gather(temb, [lane, jnp.broadcast_to(p, (16,))])

            for t in range(NT):
                for li in range(HALO, 2 * HALO - t - 1):
                    sa = sT[li - HALO]
                    sb = sT[li + t + 1 - HALO]
                    off = (t * NBIN + sa * 20 + sb) * D
                    v = trow(li) + trow(li + t + 1)
                    plsc.addupdate(acc.at[pl.ds(pl.multiple_of(off, D), D)],
                                   v)

        pltpu.sync_copy(acc, out_hbm.at[wid])

    return sc_hist


def kernel(query_seq, emb, k):
    L = query_seq.shape[0]
    D = emb.shape[-1]
    seq32 = query_seq.astype(jnp.int32)
    embt = emb.T
    partials = _build_sc_hist(L, D)(seq32, embt)
    hist = partials.sum(axis=0).reshape(NT, NBIN, D)
    t = jnp.arange(NT)
    n = (L - t - 1).astype(jnp.float32)
    gate = (t <= k).astype(jnp.float32)
    out = hist * (0.5 * gate / n)[:, None, None]
    return out.reshape(NT, 20, 20, D)
